# baseline (speedup 1.0000x reference)
import functools

import jax
import jax.numpy as jnp
from jax import lax
from jax.experimental import pallas as pl
from jax.experimental.pallas import tpu as pltpu
from jax.experimental.pallas import tpu_sc as plsc

NC = 2
NS = 16
NW = NC * NS
LANES = 16

CHUNK = 16


def _deg_body(src_hbm, out_hbm, src_v, hist_v, sem, *, epw):
    c = lax.axis_index("c")
    s = lax.axis_index("s")
    wid = s * NC + c
    base = wid * epw

    pltpu.sync_copy(src_hbm.at[pl.ds(base, epw)], src_v)

    zeros = jnp.zeros((LANES,), jnp.float32)
    n_bins = hist_v.shape[0]

    def zero_body(i, _):
        hist_v[pl.ds(i * LANES, LANES)] = zeros
        return 0

    lax.fori_loop(0, n_bins // LANES, zero_body, 0)

    ones = jnp.ones((LANES,), jnp.float32)

    def body(i, _):
        idx = src_v[pl.ds(i * LANES, LANES)]
        plsc.addupdate_scatter(hist_v, [idx], ones)
        return 0

    lax.fori_loop(0, epw // LANES, body, 0)

    pltpu.sync_copy(hist_v, out_hbm.at[wid])


def _degree_partials(src, npad):
    e = src.shape[0]
    epw = e // NW
    mesh = plsc.VectorSubcoreMesh(core_axis_name="c", subcore_axis_name="s")
    kern = functools.partial(
        pl.kernel,
        out_type=jax.ShapeDtypeStruct((NW, npad), jnp.float32),
        mesh=mesh,
        scratch_types=[
            pltpu.VMEM((epw,), jnp.int32),
            pltpu.VMEM((npad,), jnp.float32),
            pltpu.SemaphoreType.DMA,
        ],
        compiler_params=pltpu.CompilerParams(needs_layout_passes=False),
    )(functools.partial(_deg_body, epw=epw))
    return kern(src)


def _norm_col(dp_block):
    ones = jnp.ones((dp_block.shape[0], 1), jnp.float32)
    deg = lax.dot_general(dp_block, ones, (((0,), (0,)), ((), ())),
                          preferred_element_type=jnp.float32)
    return lax.rsqrt(jnp.maximum(deg, 1.0))


def _feat_body(dp_ref, f_ref, feat_ref, norm_ref):
    norm = _norm_col(dp_ref[...])
    norm_ref[...] = norm
    feat_ref[...] = f_ref[...] * norm


def _scaled_features(deg_part, features_pad):
    npad, d = features_pad.shape
    blk = 1024
    grid = npad // blk
    return pl.pallas_call(
        _feat_body,
        grid=(grid,),
        in_specs=[
            pl.BlockSpec((NW, blk), lambda i: (0, i)),
            pl.BlockSpec((blk, d), lambda i: (i, 0)),
        ],
        out_specs=[
            pl.BlockSpec((blk, d), lambda i: (i, 0)),
            pl.BlockSpec((blk, 1), lambda i: (i, 0)),
        ],
        out_shape=[
            jax.ShapeDtypeStruct((npad, d), jnp.float32),
            jax.ShapeDtypeStruct((npad, 1), jnp.float32),
        ],
    )(deg_part, features_pad)


def _agg_body(feat_hbm, src_hbm, dst_hbm, out_hbm,
              srcc0_v, srcc1_v, srcc2_v, srcc3_v,
              dstc0_v, dstc1_v, dstc2_v, dstc3_v, rows_v,
              acc_sh,
              gsem0, gsem1, gsem2, gsem3,
              isem0, isem1, isem2, isem3,
              ssem0, ssem1, ssem2, ssem3, *, npad, epw, e):
    del e
    c = lax.axis_index("c")
    s = lax.axis_index("s")
    wid = s * NC + c
    base = wid * epw
    nchunk = epw // CHUNK
    assert nchunk % 4 == 1 and nchunk >= 9

    zrows = rows_v.shape[0]
    zeros = jnp.zeros((LANES,), jnp.float32)
    d = rows_v.shape[1]

    def zero_body(i, _):
        r = i // (d // LANES)
        col = (i % (d // LANES)) * LANES
        rows_v[r, pl.ds(col, LANES)] = zeros
        return 0

    lax.fori_loop(0, zrows * (d // LANES), zero_body, 0)
    rows_per_tile = npad // NS
    for k in range(rows_per_tile // zrows):
        pltpu.sync_copy(rows_v,
                        acc_sh.at[pl.ds(s * rows_per_tile + k * zrows, zrows)])
    plsc.subcore_barrier()

    srcc = (srcc0_v, srcc1_v, srcc2_v, srcc3_v)
    dstc = (dstc0_v, dstc1_v, dstc2_v, dstc3_v)
    gsem = (gsem0, gsem1, gsem2, gsem3)
    isem = (isem0, isem1, isem2, isem3)
    ssem = (ssem0, ssem1, ssem2, ssem3)

    def start_idx(ci, b):
        off = pl.multiple_of(ci * CHUNK, 8)
        pltpu.async_copy(src_hbm.at[pl.ds(base + off, CHUNK)],
                         srcc[b], isem[b])
        pltpu.async_copy(dst_hbm.at[pl.ds(base + off, CHUNK)],
                         dstc[b], isem[b])

    def wait_idx(b):
        pltpu.make_async_copy(src_hbm.at[pl.ds(0, CHUNK)],
                              srcc[b], isem[b]).wait()
        pltpu.make_async_copy(dst_hbm.at[pl.ds(0, CHUNK)],
                              dstc[b], isem[b]).wait()

    def rows_at(b):
        return rows_v.at[pl.ds(b * CHUNK, CHUNK)]

    def start_gather(b):
        pltpu.async_copy(feat_hbm.at[srcc[b]], rows_at(b), gsem[b])

    def wait_gather(b):
        pltpu.make_async_copy(feat_hbm.at[srcc[b]], rows_at(b),
                              gsem[b]).wait()

    def start_scatter(b):
        pltpu.async_copy(rows_at(b), acc_sh.at[dstc[b]], ssem[b], add=True)

    def wait_scatter(b):
        pltpu.make_async_copy(rows_at(b), acc_sh.at[dstc[b]], ssem[b]).wait()

    start_idx(0, 0)
    start_idx(1, 1)
    start_idx(2, 2)
    wait_idx(0)
    start_gather(0)
    wait_idx(1)
    start_gather(1)

    def step(ci, j, do_gather, do_idx, first):
        jg = (j + 2) % 4
        jp = (j + 3) % 4
        if do_gather:
            wait_idx(jg)
            start_gather(jg)
        wait_gather(j)
        start_scatter(j)
        if not first:
            wait_scatter(jp)
        if do_idx:
            start_idx(ci + 3, jp)

    def group_body(g, _):
        ci0 = 4 * g
        for j in range(4):
            ci = ci0 + j
            jg = (j + 2) % 4
            jp = (j + 3) % 4
            wait_idx(jg)
            start_gather(jg)
            wait_gather(j)
            start_scatter(j)

            @pl.when(ci >= 1)
            def _():
                wait_scatter(jp)

            start_idx(ci + 3, jp)
        return 0

    ngroups = (nchunk - 5) // 4
    lax.fori_loop(0, ngroups, group_body, 0)

    for t in range(5):
        ci = nchunk - 5 + t
        step(ci, ci % 4, do_gather=(t < 3), do_idx=(t < 2), first=False)
    wait_scatter((nchunk - 1) % 4)

    plsc.subcore_barrier()

    for k in range(rows_per_tile // zrows):
        r0 = s * rows_per_tile + k * zrows
        pltpu.sync_copy(acc_sh.at[pl.ds(r0, zrows)],
                        out_hbm.at[c, pl.ds(r0, zrows)])


def _aggregate_partials(feat, src, dst, npad):
    _, d = feat.shape
    e = src.shape[0]
    epw = e // NW
    mesh = plsc.VectorSubcoreMesh(core_axis_name="c", subcore_axis_name="s")
    kern = functools.partial(
        pl.kernel,
        out_type=jax.ShapeDtypeStruct((NC, npad, d), jnp.float32),
        mesh=mesh,
        scratch_types=[
            pltpu.VMEM((CHUNK,), jnp.int32),
            pltpu.VMEM((CHUNK,), jnp.int32),
            pltpu.VMEM((CHUNK,), jnp.int32),
            pltpu.VMEM((CHUNK,), jnp.int32),
            pltpu.VMEM((CHUNK,), jnp.int32),
            pltpu.VMEM((CHUNK,), jnp.int32),
            pltpu.VMEM((CHUNK,), jnp.int32),
            pltpu.VMEM((CHUNK,), jnp.int32),
            pltpu.VMEM((4 * CHUNK, d), jnp.float32),
            pltpu.VMEM_SHARED((npad, d), jnp.float32),
        ] + [pltpu.SemaphoreType.DMA] * 12,
        compiler_params=pltpu.CompilerParams(needs_layout_passes=False),
    )(functools.partial(_agg_body, npad=npad, epw=epw, e=e))
    return kern(feat, src, dst)


def _head_body(p0_ref, p1_ref, norm_ref, wgc_ref, bgc_ref, wl_ref, blin_ref,
               out_ref, out2_ref):
    agg = p0_ref[0] + p1_ref[0]
    rst = jnp.dot(agg, wgc_ref[...], preferred_element_type=jnp.float32)
    rst = rst * norm_ref[...] + bgc_ref[...]
    h = jnp.maximum(rst, 0.0)
    out = lax.dot_general(h, wl_ref[...], (((1,), (1,)), ((), ())),
                          preferred_element_type=jnp.float32) + blin_ref[...]
    out_ref[...] = out
    out2_ref[...] = out


def _head(parts, norm, w_gc, b_gc, w_lin, b_lin, n):
    _, npad, d = parts.shape
    d_out = w_lin.shape[0]
    blk = 1000
    grid = n // blk
    return pl.pallas_call(
        _head_body,
        grid=(grid,),
        in_specs=[
            pl.BlockSpec((1, blk, d), lambda i: (0, i, 0)),
            pl.BlockSpec((1, blk, d), lambda i: (1, i, 0)),
            pl.BlockSpec((blk, 1), lambda i: (i, 0)),
            pl.BlockSpec((d, d), lambda i: (0, 0)),
            pl.BlockSpec((1, d), lambda i: (0, 0)),
            pl.BlockSpec((d_out, d), lambda i: (0, 0)),
            pl.BlockSpec((1, d_out), lambda i: (0, 0)),
        ],
        out_specs=[pl.BlockSpec((blk, d_out), lambda i: (i, 0)),
                   pl.BlockSpec((blk, d_out), lambda i: (i, 0))],
        out_shape=[jax.ShapeDtypeStruct((n, d_out), jnp.float32),
                   jax.ShapeDtypeStruct((n, d_out), jnp.float32)],
    )(parts, parts, norm, w_gc, b_gc, w_lin, b_lin)


def kernel(n_subgraph, e_subgraph, to_fetch, features, W_gc, b_gc, W_lin, b_lin):
    n = n_subgraph.shape[0]

    npad = ((n + NS * 128 - 1) // (NS * 128)) * (NS * 128)
    features_pad = jnp.pad(features, ((0, npad - n), (0, 0)))
    src = e_subgraph[0]
    dst = e_subgraph[1]
    deg_part = _degree_partials(src, npad)
    feat, norm = _scaled_features(deg_part, features_pad)
    parts = _aggregate_partials(feat, src, dst, npad)
    out, out2 = _head(parts, norm, W_gc, b_gc.reshape(1, -1),
                      W_lin, b_lin.reshape(1, -1), n)
    return (out, out2)


# speedup vs baseline: 2.1651x; 2.1651x over previous
import functools

import jax
import jax.numpy as jnp
from jax import lax
from jax.experimental import pallas as pl
from jax.experimental.pallas import tpu as pltpu
from jax.experimental.pallas import tpu_sc as plsc

NC = 2
NS = 16
NW = NC * NS
LANES = 16

CHUNK = 96


def _deg_body(edges_hbm, out_hbm, src_v, hist_v, sem, *, epw):
    c = lax.axis_index("c")
    s = lax.axis_index("s")
    wid = s * NC + c
    base = wid * epw

    pltpu.sync_copy(edges_hbm.at[pl.ds(base, epw)], src_v)

    zeros = jnp.zeros((LANES,), jnp.float32)
    n_bins = hist_v.shape[0]

    def zero_body(i, _):
        hist_v[pl.ds(i * LANES, LANES)] = zeros
        return 0

    lax.fori_loop(0, n_bins // LANES, zero_body, 0)

    ones = jnp.ones((LANES,), jnp.float32)

    def body(i, _):
        idx = src_v[pl.ds(i * LANES, LANES)]
        plsc.addupdate_scatter(hist_v, [idx], ones)
        return 0

    lax.fori_loop(0, epw // LANES, body, 0)

    pltpu.sync_copy(hist_v, out_hbm.at[wid])


def _degree_partials(edges, npad):
    e = edges.shape[0] // 2
    epw = e // NW
    mesh = plsc.VectorSubcoreMesh(core_axis_name="c", subcore_axis_name="s")
    kern = functools.partial(
        pl.kernel,
        out_type=jax.ShapeDtypeStruct((NW, npad), jnp.float32),
        mesh=mesh,
        scratch_types=[
            pltpu.VMEM((epw,), jnp.int32),
            pltpu.VMEM((npad,), jnp.float32),
            pltpu.SemaphoreType.DMA,
        ],
        compiler_params=pltpu.CompilerParams(needs_layout_passes=False),
    )(functools.partial(_deg_body, epw=epw))
    return kern(edges)


def _norm_col(dp_block):
    ones = jnp.ones((dp_block.shape[0], 1), jnp.float32)
    deg = lax.dot_general(dp_block, ones, (((0,), (0,)), ((), ())),
                          preferred_element_type=jnp.float32)
    return lax.rsqrt(jnp.maximum(deg, 1.0))


def _feat_body(dp_ref, f_ref, feat_ref, norm_ref):
    norm = _norm_col(dp_ref[...])
    norm_ref[...] = norm
    feat_ref[...] = f_ref[...] * norm


def _scaled_features(deg_part, features_pad):
    npad, d = features_pad.shape
    blk = 2048
    grid = npad // blk
    return pl.pallas_call(
        _feat_body,
        grid=(grid,),
        in_specs=[
            pl.BlockSpec((NW, blk), lambda i: (0, i)),
            pl.BlockSpec((blk, d), lambda i: (i, 0)),
        ],
        out_specs=[
            pl.BlockSpec((blk, d), lambda i: (i, 0)),
            pl.BlockSpec((blk, 1), lambda i: (i, 0)),
        ],
        out_shape=[
            jax.ShapeDtypeStruct((npad, d), jnp.float32),
            jax.ShapeDtypeStruct((npad, 1), jnp.float32),
        ],
    )(deg_part, features_pad)


def _agg_body(feat_hbm, edges_hbm, out_hbm,
              src_v, dstc0_v, dstc1_v, dstc2_v, dstt_v, rows_v,
              acc_sh,
              gsem0, gsem1, gsem2,
              isem0, isem1, isem2,
              ssem0, ssem1, ssem2, *, npad, epw, e):
    c = lax.axis_index("c")
    s = lax.axis_index("s")
    wid = s * NC + c
    base = wid * epw
    nchunk = epw // CHUNK
    tail = epw - nchunk * CHUNK
    assert (nchunk - 2) % 3 == 0 and nchunk >= 5
    assert tail % 8 == 0 and 0 < tail <= CHUNK
    d = rows_v.shape[1]

    zrows = rows_v.shape[0]
    zeros = jnp.zeros((LANES,), jnp.float32)

    def zero_body(i, _):
        r = i // (d // LANES)
        col = (i % (d // LANES)) * LANES
        rows_v[r, pl.ds(col, LANES)] = zeros
        return 0

    lax.fori_loop(0, zrows * (d // LANES), zero_body, 0)
    rows_per_tile = npad // NS
    r0 = s * rows_per_tile
    off = 0
    while off < rows_per_tile:
        nrows = min(zrows, rows_per_tile - off)
        pltpu.sync_copy(rows_v.at[pl.ds(0, nrows)],
                        acc_sh.at[pl.ds(r0 + off, nrows)])
        off += nrows
    plsc.subcore_barrier()

    pltpu.sync_copy(edges_hbm.at[pl.ds(base, epw)], src_v)

    dstc = (dstc0_v, dstc1_v, dstc2_v)
    gsem = (gsem0, gsem1, gsem2)
    isem = (isem0, isem1, isem2)
    ssem = (ssem0, ssem1, ssem2)

    def start_idx(ci, b):
        off = pl.multiple_of(ci * CHUNK, 8)
        pltpu.async_copy(edges_hbm.at[pl.ds(e + base + off, CHUNK)],
                         dstc[b], isem[b])

    def wait_idx(b):
        pltpu.make_async_copy(edges_hbm.at[pl.ds(0, CHUNK)],
                              dstc[b], isem[b]).wait()

    def rows_at(b):
        return rows_v.at[pl.ds(b * CHUNK, CHUNK)]

    def start_gather(ci, b):
        off = pl.multiple_of(ci * CHUNK, 8)
        pltpu.async_copy(feat_hbm.at[src_v.at[pl.ds(off, CHUNK)]],
                         rows_at(b), gsem[b])

    def wait_gather(b):
        pltpu.make_async_copy(feat_hbm.at[src_v.at[pl.ds(0, CHUNK)]],
                              rows_at(b), gsem[b]).wait()

    def start_scatter(b):
        pltpu.async_copy(rows_at(b), acc_sh.at[dstc[b]], ssem[b], add=True)

    def wait_scatter(b):
        pltpu.make_async_copy(rows_at(b), acc_sh.at[dstc[b]], ssem[b]).wait()

    start_idx(0, 0)
    start_gather(0, 0)
    start_idx(1, 1)
    start_gather(1, 1)

    def group_body(g, _):
        ci0 = 3 * g
        for j in range(3):
            ci = ci0 + j
            jp = (j + 2) % 3
            wait_idx(j)
            wait_gather(j)
            start_scatter(j)

            @pl.when(ci >= 1)
            def _():
                wait_scatter(jp)

            start_idx(ci + 2, jp)
            start_gather(ci + 2, jp)
        return 0

    ngroups = (nchunk - 2) // 3
    lax.fori_loop(0, ngroups, group_body, 0)

    for t in range(2):
        j = (nchunk - 2 + t) % 3
        jp = (j + 2) % 3
        wait_idx(j)
        wait_gather(j)
        start_scatter(j)
        wait_scatter(jp)
    wait_scatter((nchunk - 1) % 3)

    toff = pl.multiple_of(nchunk * CHUNK, 8)
    pltpu.async_copy(edges_hbm.at[pl.ds(e + base + toff, tail)],
                     dstt_v, isem0)
    pltpu.async_copy(feat_hbm.at[src_v.at[pl.ds(toff, tail)]],
                     rows_v.at[pl.ds(0, tail)], gsem0)
    pltpu.make_async_copy(edges_hbm.at[pl.ds(0, tail)], dstt_v, isem0).wait()
    pltpu.make_async_copy(feat_hbm.at[src_v.at[pl.ds(0, tail)]],
                          rows_v.at[pl.ds(0, tail)], gsem0).wait()
    pltpu.sync_copy(rows_v.at[pl.ds(0, tail)], acc_sh.at[dstt_v], add=True)

    plsc.subcore_barrier()

    off = 0
    while off < rows_per_tile:
        nrows = min(zrows, rows_per_tile - off)
        pltpu.sync_copy(acc_sh.at[pl.ds(r0 + off, nrows)],
                        out_hbm.at[c, pl.ds(r0 + off, nrows)])
        off += nrows


def _aggregate_partials(feat, edges, npad):
    _, d = feat.shape
    e = edges.shape[0] // 2
    epw = e // NW
    mesh = plsc.VectorSubcoreMesh(core_axis_name="c", subcore_axis_name="s")
    kern = functools.partial(
        pl.kernel,
        out_type=jax.ShapeDtypeStruct((NC, npad, d), jnp.float32),
        mesh=mesh,
        scratch_types=[
            pltpu.VMEM((epw,), jnp.int32),
            pltpu.VMEM((CHUNK,), jnp.int32),
            pltpu.VMEM((CHUNK,), jnp.int32),
            pltpu.VMEM((CHUNK,), jnp.int32),
            pltpu.VMEM((epw - (epw // CHUNK) * CHUNK,), jnp.int32),
            pltpu.VMEM((3 * CHUNK, d), jnp.float32),
            pltpu.VMEM_SHARED((npad, d), jnp.float32),
        ] + [pltpu.SemaphoreType.DMA] * 9,
        compiler_params=pltpu.CompilerParams(needs_layout_passes=False),
    )(functools.partial(_agg_body, npad=npad, epw=epw, e=e))
    return kern(feat, edges)


def _head_body(p0_ref, p1_ref, norm_ref, wgc_ref, bgc_ref, wl_ref, blin_ref,
               out_ref, out2_ref):
    agg = p0_ref[0] + p1_ref[0]
    rst = jnp.dot(agg, wgc_ref[...], preferred_element_type=jnp.float32)
    rst = rst * norm_ref[...] + bgc_ref[...]
    h = jnp.maximum(rst, 0.0)
    out = lax.dot_general(h, wl_ref[...], (((1,), (1,)), ((), ())),
                          preferred_element_type=jnp.float32) + blin_ref[...]
    out_ref[...] = out
    out2_ref[...] = out


def _head(parts, norm, w_gc, b_gc, w_lin, b_lin, n):
    _, npad, d = parts.shape
    d_out = w_lin.shape[0]
    blk = 2000
    grid = n // blk
    return pl.pallas_call(
        _head_body,
        grid=(grid,),
        in_specs=[
            pl.BlockSpec((1, blk, d), lambda i: (0, i, 0)),
            pl.BlockSpec((1, blk, d), lambda i: (1, i, 0)),
            pl.BlockSpec((blk, 1), lambda i: (i, 0)),
            pl.BlockSpec((d, d), lambda i: (0, 0)),
            pl.BlockSpec((1, d), lambda i: (0, 0)),
            pl.BlockSpec((d_out, d), lambda i: (0, 0)),
            pl.BlockSpec((1, d_out), lambda i: (0, 0)),
        ],
        out_specs=[pl.BlockSpec((blk, d_out), lambda i: (i, 0)),
                   pl.BlockSpec((blk, d_out), lambda i: (i, 0))],
        out_shape=[jax.ShapeDtypeStruct((n, d_out), jnp.float32),
                   jax.ShapeDtypeStruct((n, d_out), jnp.float32)],
    )(parts, parts, norm, w_gc, b_gc, w_lin, b_lin)


def kernel(n_subgraph, e_subgraph, to_fetch, features, W_gc, b_gc, W_lin, b_lin):
    n = n_subgraph.shape[0]

    npad = ((n + NS * 128 - 1) // (NS * 128)) * (NS * 128)
    features_pad = jnp.pad(features, ((0, npad - n), (0, 0)))
    edges_flat = e_subgraph.reshape(-1)
    deg_part = _degree_partials(edges_flat, npad)
    feat, norm = _scaled_features(deg_part, features_pad)
    parts = _aggregate_partials(feat, edges_flat, npad)
    out, out2 = _head(parts, norm, W_gc, b_gc.reshape(1, -1),
                      W_lin, b_lin.reshape(1, -1), n)
    return (out, out2)


# speedup vs baseline: 2.1686x; 1.0016x over previous
import functools

import jax
import jax.numpy as jnp
from jax import lax
from jax.experimental import pallas as pl
from jax.experimental.pallas import tpu as pltpu
from jax.experimental.pallas import tpu_sc as plsc

NC = 2
NS = 16
NW = NC * NS
LANES = 16

CHUNK = 96


def _deg_body(edges_hbm, out_hbm, src_v, hist_v, sem, *, epw):
    c = lax.axis_index("c")
    s = lax.axis_index("s")
    wid = s * NC + c
    base = wid * epw

    pltpu.sync_copy(edges_hbm.at[pl.ds(base, epw)], src_v)

    zeros = jnp.zeros((LANES,), jnp.float32)
    n_bins = hist_v.shape[0]

    def zero_body(i, _):
        hist_v[pl.ds(i * LANES, LANES)] = zeros
        return 0

    lax.fori_loop(0, n_bins // LANES, zero_body, 0)

    ones = jnp.ones((LANES,), jnp.float32)

    def body(i, _):
        for u in range(4):
            idx = src_v[pl.ds((i * 4 + u) * LANES, LANES)]
            plsc.addupdate_scatter(hist_v, [idx], ones)
        return 0

    lax.fori_loop(0, epw // (4 * LANES), body, 0)
    for i in range(epw // (4 * LANES) * 4, epw // LANES):
        idx = src_v[pl.ds(i * LANES, LANES)]
        plsc.addupdate_scatter(hist_v, [idx], ones)

    pltpu.sync_copy(hist_v, out_hbm.at[wid])


def _degree_partials(edges, npad):
    e = edges.shape[0] // 2
    epw = e // NW
    mesh = plsc.VectorSubcoreMesh(core_axis_name="c", subcore_axis_name="s")
    kern = functools.partial(
        pl.kernel,
        out_type=jax.ShapeDtypeStruct((NW, npad), jnp.float32),
        mesh=mesh,
        scratch_types=[
            pltpu.VMEM((epw,), jnp.int32),
            pltpu.VMEM((npad,), jnp.float32),
            pltpu.SemaphoreType.DMA,
        ],
        compiler_params=pltpu.CompilerParams(needs_layout_passes=False),
    )(functools.partial(_deg_body, epw=epw))
    return kern(edges)


def _norm_col(dp_block):
    ones = jnp.ones((dp_block.shape[0], 1), jnp.float32)
    deg = lax.dot_general(dp_block, ones, (((0,), (0,)), ((), ())),
                          preferred_element_type=jnp.float32)
    return lax.rsqrt(jnp.maximum(deg, 1.0))


def _feat_body(dp_ref, f_ref, feat_ref, norm_ref):
    norm = _norm_col(dp_ref[...])
    norm_ref[...] = norm
    feat_ref[...] = f_ref[...] * norm


def _scaled_features(deg_part, features_pad):
    npad, d = features_pad.shape
    blk = 2048
    grid = npad // blk
    return pl.pallas_call(
        _feat_body,
        grid=(grid,),
        in_specs=[
            pl.BlockSpec((NW, blk), lambda i: (0, i)),
            pl.BlockSpec((blk, d), lambda i: (i, 0)),
        ],
        out_specs=[
            pl.BlockSpec((blk, d), lambda i: (i, 0)),
            pl.BlockSpec((blk, 1), lambda i: (i, 0)),
        ],
        out_shape=[
            jax.ShapeDtypeStruct((npad, d), jnp.float32),
            jax.ShapeDtypeStruct((npad, 1), jnp.float32),
        ],
    )(deg_part, features_pad)


def _agg_body(feat_hbm, edges_hbm, out_hbm,
              src_v, dstc0_v, dstc1_v, dstc2_v, dstt_v, rows_v,
              acc_sh,
              gsem0, gsem1, gsem2,
              isem0, isem1, isem2,
              ssem0, ssem1, ssem2, *, npad, epw, e):
    c = lax.axis_index("c")
    s = lax.axis_index("s")
    wid = s * NC + c
    base = wid * epw
    nchunk = epw // CHUNK
    tail = epw - nchunk * CHUNK
    assert (nchunk - 2) % 3 == 0 and nchunk >= 5
    assert tail % 8 == 0 and 0 < tail <= CHUNK
    d = rows_v.shape[1]

    zrows = rows_v.shape[0]
    zeros = jnp.zeros((LANES,), jnp.float32)

    def zero_body(i, _):
        r = i // (d // LANES)
        col = (i % (d // LANES)) * LANES
        rows_v[r, pl.ds(col, LANES)] = zeros
        return 0

    lax.fori_loop(0, zrows * (d // LANES), zero_body, 0)
    rows_per_tile = npad // NS
    r0 = s * rows_per_tile
    off = 0
    while off < rows_per_tile:
        nrows = min(zrows, rows_per_tile - off)
        pltpu.sync_copy(rows_v.at[pl.ds(0, nrows)],
                        acc_sh.at[pl.ds(r0 + off, nrows)])
        off += nrows
    plsc.subcore_barrier()

    pltpu.sync_copy(edges_hbm.at[pl.ds(base, epw)], src_v)

    dstc = (dstc0_v, dstc1_v, dstc2_v)
    gsem = (gsem0, gsem1, gsem2)
    isem = (isem0, isem1, isem2)
    ssem = (ssem0, ssem1, ssem2)

    def start_idx(ci, b):
        off = pl.multiple_of(ci * CHUNK, 8)
        pltpu.async_copy(edges_hbm.at[pl.ds(e + base + off, CHUNK)],
                         dstc[b], isem[b])

    def wait_idx(b):
        pltpu.make_async_copy(edges_hbm.at[pl.ds(0, CHUNK)],
                              dstc[b], isem[b]).wait()

    def rows_at(b):
        return rows_v.at[pl.ds(b * CHUNK, CHUNK)]

    def start_gather(ci, b):
        off = pl.multiple_of(ci * CHUNK, 8)
        pltpu.async_copy(feat_hbm.at[src_v.at[pl.ds(off, CHUNK)]],
                         rows_at(b), gsem[b])

    def wait_gather(b):
        pltpu.make_async_copy(feat_hbm.at[src_v.at[pl.ds(0, CHUNK)]],
                              rows_at(b), gsem[b]).wait()

    def start_scatter(b):
        pltpu.async_copy(rows_at(b), acc_sh.at[dstc[b]], ssem[b], add=True)

    def wait_scatter(b):
        pltpu.make_async_copy(rows_at(b), acc_sh.at[dstc[b]], ssem[b]).wait()

    start_idx(0, 0)
    start_gather(0, 0)
    start_idx(1, 1)
    start_gather(1, 1)

    def group_body(g, _):
        ci0 = 3 * g
        for j in range(3):
            ci = ci0 + j
            jp = (j + 2) % 3
            wait_idx(j)
            wait_gather(j)
            start_scatter(j)

            @pl.when(ci >= 1)
            def _():
                wait_scatter(jp)

            start_idx(ci + 2, jp)
            start_gather(ci + 2, jp)
        return 0

    ngroups = (nchunk - 2) // 3
    lax.fori_loop(0, ngroups, group_body, 0)

    for t in range(2):
        j = (nchunk - 2 + t) % 3
        jp = (j + 2) % 3
        wait_idx(j)
        wait_gather(j)
        start_scatter(j)
        wait_scatter(jp)
    wait_scatter((nchunk - 1) % 3)

    toff = pl.multiple_of(nchunk * CHUNK, 8)
    pltpu.async_copy(edges_hbm.at[pl.ds(e + base + toff, tail)],
                     dstt_v, isem0)
    pltpu.async_copy(feat_hbm.at[src_v.at[pl.ds(toff, tail)]],
                     rows_v.at[pl.ds(0, tail)], gsem0)
    pltpu.make_async_copy(edges_hbm.at[pl.ds(0, tail)], dstt_v, isem0).wait()
    pltpu.make_async_copy(feat_hbm.at[src_v.at[pl.ds(0, tail)]],
                          rows_v.at[pl.ds(0, tail)], gsem0).wait()
    pltpu.sync_copy(rows_v.at[pl.ds(0, tail)], acc_sh.at[dstt_v], add=True)

    plsc.subcore_barrier()

    off = 0
    while off < rows_per_tile:
        nrows = min(zrows, rows_per_tile - off)
        pltpu.sync_copy(acc_sh.at[pl.ds(r0 + off, nrows)],
                        out_hbm.at[c, pl.ds(r0 + off, nrows)])
        off += nrows


def _aggregate_partials(feat, edges, npad):
    _, d = feat.shape
    e = edges.shape[0] // 2
    epw = e // NW
    mesh = plsc.VectorSubcoreMesh(core_axis_name="c", subcore_axis_name="s")
    kern = functools.partial(
        pl.kernel,
        out_type=jax.ShapeDtypeStruct((NC, npad, d), jnp.float32),
        mesh=mesh,
        scratch_types=[
            pltpu.VMEM((epw,), jnp.int32),
            pltpu.VMEM((CHUNK,), jnp.int32),
            pltpu.VMEM((CHUNK,), jnp.int32),
            pltpu.VMEM((CHUNK,), jnp.int32),
            pltpu.VMEM((epw - (epw // CHUNK) * CHUNK,), jnp.int32),
            pltpu.VMEM((3 * CHUNK, d), jnp.float32),
            pltpu.VMEM_SHARED((npad, d), jnp.float32),
        ] + [pltpu.SemaphoreType.DMA] * 9,
        compiler_params=pltpu.CompilerParams(needs_layout_passes=False),
    )(functools.partial(_agg_body, npad=npad, epw=epw, e=e))
    return kern(feat, edges)


def _head_body(p_ref, norm_ref, wgc_ref, bgc_ref, wl_ref, blin_ref,
               out_ref, out2_ref):
    agg = p_ref[0] + p_ref[1]
    rst = jnp.dot(agg, wgc_ref[...], preferred_element_type=jnp.float32)
    rst = rst * norm_ref[...] + bgc_ref[...]
    h = jnp.maximum(rst, 0.0)
    out = lax.dot_general(h, wl_ref[...], (((1,), (1,)), ((), ())),
                          preferred_element_type=jnp.float32) + blin_ref[...]
    out_ref[...] = out
    out2_ref[...] = out


def _head(parts, norm, w_gc, b_gc, w_lin, b_lin, n):
    _, npad, d = parts.shape
    d_out = w_lin.shape[0]
    blk = 2000
    grid = n // blk
    return pl.pallas_call(
        _head_body,
        grid=(grid,),
        in_specs=[
            pl.BlockSpec((2, blk, d), lambda i: (0, i, 0)),
            pl.BlockSpec((blk, 1), lambda i: (i, 0)),
            pl.BlockSpec((d, d), lambda i: (0, 0)),
            pl.BlockSpec((1, d), lambda i: (0, 0)),
            pl.BlockSpec((d_out, d), lambda i: (0, 0)),
            pl.BlockSpec((1, d_out), lambda i: (0, 0)),
        ],
        out_specs=[pl.BlockSpec((blk, d_out), lambda i: (i, 0)),
                   pl.BlockSpec((blk, d_out), lambda i: (i, 0))],
        out_shape=[jax.ShapeDtypeStruct((n, d_out), jnp.float32),
                   jax.ShapeDtypeStruct((n, d_out), jnp.float32)],
    )(parts, norm, w_gc, b_gc, w_lin, b_lin)


def kernel(n_subgraph, e_subgraph, to_fetch, features, W_gc, b_gc, W_lin, b_lin):
    n = n_subgraph.shape[0]

    npad = ((n + NS * 128 - 1) // (NS * 128)) * (NS * 128)
    features_pad = jnp.pad(features, ((0, npad - n), (0, 0)))
    edges_flat = e_subgraph.reshape(-1)
    deg_part = _degree_partials(edges_flat, npad)
    feat, norm = _scaled_features(deg_part, features_pad)
    parts = _aggregate_partials(feat, edges_flat, npad)
    out, out2 = _head(parts, norm, W_gc, b_gc.reshape(1, -1),
                      W_lin, b_lin.reshape(1, -1), n)
    return (out, out2)


# speedup vs baseline: 2.2613x; 1.0428x over previous
import functools

import jax
import jax.numpy as jnp
from jax import lax
from jax.experimental import pallas as pl
from jax.experimental.pallas import tpu as pltpu
from jax.experimental.pallas import tpu_sc as plsc

NC = 2
NS = 16
NW = NC * NS
LANES = 16

CHUNK = 96


def _deg_body(edges_hbm, out_hbm, src_v, hist_v, sem, *, epw):
    c = lax.axis_index("c")
    s = lax.axis_index("s")
    wid = s * NC + c
    base = wid * epw

    pltpu.sync_copy(edges_hbm.at[pl.ds(base, epw)], src_v)

    zeros = jnp.zeros((LANES,), jnp.float32)
    n_bins = hist_v.shape[0]

    def zero_body(i, _):
        hist_v[pl.ds(i * LANES, LANES)] = zeros
        return 0

    lax.fori_loop(0, n_bins // LANES, zero_body, 0)

    ones = jnp.ones((LANES,), jnp.float32)

    def body(i, _):
        for u in range(4):
            idx = src_v[pl.ds((i * 4 + u) * LANES, LANES)]
            plsc.addupdate_scatter(hist_v, [idx], ones)
        return 0

    lax.fori_loop(0, epw // (4 * LANES), body, 0)
    for i in range(epw // (4 * LANES) * 4, epw // LANES):
        idx = src_v[pl.ds(i * LANES, LANES)]
        plsc.addupdate_scatter(hist_v, [idx], ones)

    pltpu.sync_copy(hist_v, out_hbm.at[wid])


def _degree_partials(edges, npad):
    e = edges.shape[0] // 2
    epw = e // NW
    mesh = plsc.VectorSubcoreMesh(core_axis_name="c", subcore_axis_name="s")
    kern = functools.partial(
        pl.kernel,
        out_type=jax.ShapeDtypeStruct((NW, npad), jnp.float32),
        mesh=mesh,
        scratch_types=[
            pltpu.VMEM((epw,), jnp.int32),
            pltpu.VMEM((npad,), jnp.float32),
            pltpu.SemaphoreType.DMA,
        ],
        compiler_params=pltpu.CompilerParams(needs_layout_passes=False),
    )(functools.partial(_deg_body, epw=epw))
    return kern(edges)


def _norm_col(dp_block):
    ones = jnp.ones((dp_block.shape[0], 1), jnp.float32)
    deg = lax.dot_general(dp_block, ones, (((0,), (0,)), ((), ())),
                          preferred_element_type=jnp.float32)
    return lax.rsqrt(jnp.maximum(deg, 1.0))


def _feat_body(dp_ref, f_ref, feat_ref, norm_ref):
    norm = _norm_col(dp_ref[...])
    norm_ref[...] = norm
    feat_ref[...] = f_ref[...] * norm


def _scaled_features(deg_part, features_pad):
    npad, d = features_pad.shape
    blk = 2048
    grid = npad // blk
    return pl.pallas_call(
        _feat_body,
        grid=(grid,),
        in_specs=[
            pl.BlockSpec((NW, blk), lambda i: (0, i)),
            pl.BlockSpec((blk, d), lambda i: (i, 0)),
        ],
        out_specs=[
            pl.BlockSpec((blk, d), lambda i: (i, 0)),
            pl.BlockSpec((blk, 1), lambda i: (i, 0)),
        ],
        out_shape=[
            jax.ShapeDtypeStruct((npad, d), jnp.float32),
            jax.ShapeDtypeStruct((npad, 1), jnp.float32),
        ],
    )(deg_part, features_pad)


def _agg_body(feat_hbm, edges_hbm, out_hbm,
              src_v, dstc0_v, dstc1_v, dstc2_v, dstt_v, rows_v,
              acc_sh,
              gsem0, gsem1, gsem2,
              isem0, isem1, isem2,
              ssem0, ssem1, ssem2, *, npad, epw, e):
    c = lax.axis_index("c")
    s = lax.axis_index("s")
    wid = s * NC + c
    base = wid * epw
    nchunk = epw // CHUNK
    tail = epw - nchunk * CHUNK
    assert (nchunk - 2) % 3 == 0 and nchunk >= 5
    assert tail % 8 == 0 and 0 < tail <= CHUNK
    d = rows_v.shape[1]

    pltpu.sync_copy(edges_hbm.at[pl.ds(base, epw)], src_v)

    dstc = (dstc0_v, dstc1_v, dstc2_v)
    gsem = (gsem0, gsem1, gsem2)
    isem = (isem0, isem1, isem2)
    ssem = (ssem0, ssem1, ssem2)

    def start_idx(ci, b):
        off = pl.multiple_of(ci * CHUNK, 8)
        pltpu.async_copy(edges_hbm.at[pl.ds(e + base + off, CHUNK)],
                         dstc[b], isem[b])

    def wait_idx(b):
        pltpu.make_async_copy(edges_hbm.at[pl.ds(0, CHUNK)],
                              dstc[b], isem[b]).wait()

    def rows_at(b):
        return rows_v.at[pl.ds(b * CHUNK, CHUNK)]

    def start_gather(ci, b):
        off = pl.multiple_of(ci * CHUNK, 8)
        pltpu.async_copy(feat_hbm.at[src_v.at[pl.ds(off, CHUNK)]],
                         rows_at(b), gsem[b])

    def wait_gather(b):
        pltpu.make_async_copy(feat_hbm.at[src_v.at[pl.ds(0, CHUNK)]],
                              rows_at(b), gsem[b]).wait()

    def start_scatter(b):
        pltpu.async_copy(rows_at(b), acc_sh.at[dstc[b]], ssem[b], add=True)

    def wait_scatter(b):
        pltpu.make_async_copy(rows_at(b), acc_sh.at[dstc[b]], ssem[b]).wait()

    start_idx(0, 0)
    start_gather(0, 0)
    start_idx(1, 1)
    start_gather(1, 1)

    zeros = jnp.zeros((LANES,), jnp.float32)
    zb = 2 * CHUNK

    def zero_body(i, _):
        r = zb + i // (d // LANES)
        col = (i % (d // LANES)) * LANES
        rows_v[r, pl.ds(col, LANES)] = zeros
        return 0

    lax.fori_loop(0, CHUNK * (d // LANES), zero_body, 0)
    rows_per_tile = npad // NS
    r0 = s * rows_per_tile
    off = 0
    while off < rows_per_tile:
        nrows = min(CHUNK, rows_per_tile - off)
        pltpu.sync_copy(rows_v.at[pl.ds(zb, nrows)],
                        acc_sh.at[pl.ds(r0 + off, nrows)])
        off += nrows
    plsc.subcore_barrier()

    def group_body(g, _):
        ci0 = 3 * g
        for j in range(3):
            ci = ci0 + j
            jp = (j + 2) % 3
            wait_idx(j)
            wait_gather(j)
            start_scatter(j)

            @pl.when(ci >= 1)
            def _():
                wait_scatter(jp)

            start_idx(ci + 2, jp)
            start_gather(ci + 2, jp)
        return 0

    ngroups = (nchunk - 2) // 3
    lax.fori_loop(0, ngroups, group_body, 0)

    for t in range(2):
        j = (nchunk - 2 + t) % 3
        jp = (j + 2) % 3
        wait_idx(j)
        wait_gather(j)
        start_scatter(j)
        wait_scatter(jp)
    wait_scatter((nchunk - 1) % 3)

    toff = pl.multiple_of(nchunk * CHUNK, 8)
    pltpu.async_copy(edges_hbm.at[pl.ds(e + base + toff, tail)],
                     dstt_v, isem0)
    pltpu.async_copy(feat_hbm.at[src_v.at[pl.ds(toff, tail)]],
                     rows_v.at[pl.ds(0, tail)], gsem0)
    pltpu.make_async_copy(edges_hbm.at[pl.ds(0, tail)], dstt_v, isem0).wait()
    pltpu.make_async_copy(feat_hbm.at[src_v.at[pl.ds(0, tail)]],
                          rows_v.at[pl.ds(0, tail)], gsem0).wait()
    pltpu.sync_copy(rows_v.at[pl.ds(0, tail)], acc_sh.at[dstt_v], add=True)

    plsc.subcore_barrier()

    off = 0
    while off < rows_per_tile:
        nrows = min(3 * CHUNK, rows_per_tile - off)
        pltpu.sync_copy(acc_sh.at[pl.ds(r0 + off, nrows)],
                        out_hbm.at[c, pl.ds(r0 + off, nrows)])
        off += nrows


def _aggregate_partials(feat, edges, npad):
    _, d = feat.shape
    e = edges.shape[0] // 2
    epw = e // NW
    mesh = plsc.VectorSubcoreMesh(core_axis_name="c", subcore_axis_name="s")
    kern = functools.partial(
        pl.kernel,
        out_type=jax.ShapeDtypeStruct((NC, npad, d), jnp.float32),
        mesh=mesh,
        scratch_types=[
            pltpu.VMEM((epw,), jnp.int32),
            pltpu.VMEM((CHUNK,), jnp.int32),
            pltpu.VMEM((CHUNK,), jnp.int32),
            pltpu.VMEM((CHUNK,), jnp.int32),
            pltpu.VMEM((epw - (epw // CHUNK) * CHUNK,), jnp.int32),
            pltpu.VMEM((3 * CHUNK, d), jnp.float32),
            pltpu.VMEM_SHARED((npad, d), jnp.float32),
        ] + [pltpu.SemaphoreType.DMA] * 9,
        compiler_params=pltpu.CompilerParams(needs_layout_passes=False),
    )(functools.partial(_agg_body, npad=npad, epw=epw, e=e))
    return kern(feat, edges)


def _head_body(p_ref, norm_ref, wgc_ref, bgc_ref, wl_ref, blin_ref,
               out_ref, out2_ref):
    agg = p_ref[0] + p_ref[1]
    rst = jnp.dot(agg, wgc_ref[...], preferred_element_type=jnp.float32)
    rst = rst * norm_ref[...] + bgc_ref[...]
    h = jnp.maximum(rst, 0.0)
    out = lax.dot_general(h, wl_ref[...], (((1,), (1,)), ((), ())),
                          preferred_element_type=jnp.float32) + blin_ref[...]
    out_ref[...] = out
    out2_ref[...] = out


def _head(parts, norm, w_gc, b_gc, w_lin, b_lin, n):
    _, npad, d = parts.shape
    d_out = w_lin.shape[0]
    blk = 2000
    grid = n // blk
    return pl.pallas_call(
        _head_body,
        grid=(grid,),
        in_specs=[
            pl.BlockSpec((2, blk, d), lambda i: (0, i, 0)),
            pl.BlockSpec((blk, 1), lambda i: (i, 0)),
            pl.BlockSpec((d, d), lambda i: (0, 0)),
            pl.BlockSpec((1, d), lambda i: (0, 0)),
            pl.BlockSpec((d_out, d), lambda i: (0, 0)),
            pl.BlockSpec((1, d_out), lambda i: (0, 0)),
        ],
        out_specs=[pl.BlockSpec((blk, d_out), lambda i: (i, 0)),
                   pl.BlockSpec((blk, d_out), lambda i: (i, 0))],
        out_shape=[jax.ShapeDtypeStruct((n, d_out), jnp.float32),
                   jax.ShapeDtypeStruct((n, d_out), jnp.float32)],
    )(parts, norm, w_gc, b_gc, w_lin, b_lin)


def kernel(n_subgraph, e_subgraph, to_fetch, features, W_gc, b_gc, W_lin, b_lin):
    n = n_subgraph.shape[0]

    npad = ((n + NS * 128 - 1) // (NS * 128)) * (NS * 128)
    features_pad = jnp.pad(features, ((0, npad - n), (0, 0)))
    edges_flat = e_subgraph.reshape(-1)
    deg_part = _degree_partials(edges_flat, npad)
    feat, norm = _scaled_features(deg_part, features_pad)
    parts = _aggregate_partials(feat, edges_flat, npad)
    out, out2 = _head(parts, norm, W_gc, b_gc.reshape(1, -1),
                      W_lin, b_lin.reshape(1, -1), n)
    return (out, out2)


# speedup vs baseline: 2.2887x; 1.0121x over previous
import functools

import jax
import jax.numpy as jnp
from jax import lax
from jax.experimental import pallas as pl
from jax.experimental.pallas import tpu as pltpu
from jax.experimental.pallas import tpu_sc as plsc

NC = 2
NS = 16
NW = NC * NS
LANES = 16

CHUNK = 96


def _deg_body(edges_hbm, out_hbm, src_v, hist_v, sem, *, epw):
    c = lax.axis_index("c")
    s = lax.axis_index("s")
    wid = s * NC + c
    base = wid * epw

    pltpu.sync_copy(edges_hbm.at[pl.ds(base, epw)], src_v)

    zeros = jnp.zeros((LANES,), jnp.float32)
    n_bins = hist_v.shape[0]

    def zero_body(i, _):
        hist_v[pl.ds(i * LANES, LANES)] = zeros
        return 0

    lax.fori_loop(0, n_bins // LANES, zero_body, 0)

    ones = jnp.ones((LANES,), jnp.float32)

    def body(i, _):
        for u in range(4):
            idx = src_v[pl.ds((i * 4 + u) * LANES, LANES)]
            plsc.addupdate_scatter(hist_v, [idx], ones)
        return 0

    lax.fori_loop(0, epw // (4 * LANES), body, 0)
    for i in range(epw // (4 * LANES) * 4, epw // LANES):
        idx = src_v[pl.ds(i * LANES, LANES)]
        plsc.addupdate_scatter(hist_v, [idx], ones)

    pltpu.sync_copy(hist_v, out_hbm.at[wid])


def _degree_partials(edges, npad):
    e = edges.shape[0] // 2
    epw = e // NW
    mesh = plsc.VectorSubcoreMesh(core_axis_name="c", subcore_axis_name="s")
    kern = functools.partial(
        pl.kernel,
        out_type=jax.ShapeDtypeStruct((NW, npad), jnp.float32),
        mesh=mesh,
        scratch_types=[
            pltpu.VMEM((epw,), jnp.int32),
            pltpu.VMEM((npad,), jnp.float32),
            pltpu.SemaphoreType.DMA,
        ],
        compiler_params=pltpu.CompilerParams(needs_layout_passes=False),
    )(functools.partial(_deg_body, epw=epw))
    return kern(edges)


def _norm_col(dp_block):
    ones = jnp.ones((dp_block.shape[0], 1), jnp.float32)
    deg = lax.dot_general(dp_block, ones, (((0,), (0,)), ((), ())),
                          preferred_element_type=jnp.float32)
    return lax.rsqrt(jnp.maximum(deg, 1.0))


def _feat_body(dp_ref, f_ref, feat_ref, norm_ref):
    norm = _norm_col(dp_ref[...])
    norm_ref[...] = norm
    feat_ref[...] = f_ref[...] * norm


def _scaled_features(deg_part, features_pad):
    npad, d = features_pad.shape
    blk = 5120
    grid = npad // blk
    return pl.pallas_call(
        _feat_body,
        grid=(grid,),
        in_specs=[
            pl.BlockSpec((NW, blk), lambda i: (0, i)),
            pl.BlockSpec((blk, d), lambda i: (i, 0)),
        ],
        out_specs=[
            pl.BlockSpec((blk, d), lambda i: (i, 0)),
            pl.BlockSpec((blk, 1), lambda i: (i, 0)),
        ],
        out_shape=[
            jax.ShapeDtypeStruct((npad, d), jnp.float32),
            jax.ShapeDtypeStruct((npad, 1), jnp.float32),
        ],
    )(deg_part, features_pad)


def _agg_body(feat_hbm, edges_hbm, out_hbm,
              src_v, dstc0_v, dstc1_v, dstc2_v, dstt_v, rows_v,
              acc_sh,
              gsem0, gsem1, gsem2,
              isem0, isem1, isem2,
              ssem0, ssem1, ssem2, *, npad, epw, e):
    c = lax.axis_index("c")
    s = lax.axis_index("s")
    wid = s * NC + c
    base = wid * epw
    nchunk = epw // CHUNK
    tail = epw - nchunk * CHUNK
    assert (nchunk - 2) % 3 == 0 and nchunk >= 5
    assert tail % 8 == 0 and 0 < tail <= CHUNK
    d = rows_v.shape[1]

    pltpu.sync_copy(edges_hbm.at[pl.ds(base, epw)], src_v)

    dstc = (dstc0_v, dstc1_v, dstc2_v)
    gsem = (gsem0, gsem1, gsem2)
    isem = (isem0, isem1, isem2)
    ssem = (ssem0, ssem1, ssem2)

    def start_idx(ci, b):
        off = pl.multiple_of(ci * CHUNK, 8)
        pltpu.async_copy(edges_hbm.at[pl.ds(e + base + off, CHUNK)],
                         dstc[b], isem[b])

    def wait_idx(b):
        pltpu.make_async_copy(edges_hbm.at[pl.ds(0, CHUNK)],
                              dstc[b], isem[b]).wait()

    def rows_at(b):
        return rows_v.at[pl.ds(b * CHUNK, CHUNK)]

    def start_gather(ci, b):
        off = pl.multiple_of(ci * CHUNK, 8)
        pltpu.async_copy(feat_hbm.at[src_v.at[pl.ds(off, CHUNK)]],
                         rows_at(b), gsem[b])

    def wait_gather(b):
        pltpu.make_async_copy(feat_hbm.at[src_v.at[pl.ds(0, CHUNK)]],
                              rows_at(b), gsem[b]).wait()

    def start_scatter(b):
        pltpu.async_copy(rows_at(b), acc_sh.at[dstc[b]], ssem[b], add=True)

    def wait_scatter(b):
        pltpu.make_async_copy(rows_at(b), acc_sh.at[dstc[b]], ssem[b]).wait()

    start_idx(0, 0)
    start_gather(0, 0)
    start_idx(1, 1)
    start_gather(1, 1)

    zeros = jnp.zeros((LANES,), jnp.float32)
    zb = 2 * CHUNK

    def zero_body(i, _):
        r = zb + i // (d // LANES)
        col = (i % (d // LANES)) * LANES
        rows_v[r, pl.ds(col, LANES)] = zeros
        return 0

    lax.fori_loop(0, CHUNK * (d // LANES), zero_body, 0)
    rows_per_tile = npad // NS
    r0 = s * rows_per_tile
    off = 0
    while off < rows_per_tile:
        nrows = min(CHUNK, rows_per_tile - off)
        pltpu.sync_copy(rows_v.at[pl.ds(zb, nrows)],
                        acc_sh.at[pl.ds(r0 + off, nrows)])
        off += nrows
    plsc.subcore_barrier()

    def group_body(g, _):
        ci0 = 3 * g
        for j in range(3):
            ci = ci0 + j
            jp = (j + 2) % 3
            wait_idx(j)
            wait_gather(j)
            start_scatter(j)

            @pl.when(ci >= 1)
            def _():
                wait_scatter(jp)

            start_idx(ci + 2, jp)
            start_gather(ci + 2, jp)
        return 0

    ngroups = (nchunk - 2) // 3
    lax.fori_loop(0, ngroups, group_body, 0)

    for t in range(2):
        j = (nchunk - 2 + t) % 3
        jp = (j + 2) % 3
        wait_idx(j)
        wait_gather(j)
        start_scatter(j)
        wait_scatter(jp)
    wait_scatter((nchunk - 1) % 3)

    toff = pl.multiple_of(nchunk * CHUNK, 8)
    pltpu.async_copy(edges_hbm.at[pl.ds(e + base + toff, tail)],
                     dstt_v, isem0)
    pltpu.async_copy(feat_hbm.at[src_v.at[pl.ds(toff, tail)]],
                     rows_v.at[pl.ds(0, tail)], gsem0)
    pltpu.make_async_copy(edges_hbm.at[pl.ds(0, tail)], dstt_v, isem0).wait()
    pltpu.make_async_copy(feat_hbm.at[src_v.at[pl.ds(0, tail)]],
                          rows_v.at[pl.ds(0, tail)], gsem0).wait()
    pltpu.sync_copy(rows_v.at[pl.ds(0, tail)], acc_sh.at[dstt_v], add=True)

    plsc.subcore_barrier()

    off = 0
    while off < rows_per_tile:
        nrows = min(3 * CHUNK, rows_per_tile - off)
        pltpu.sync_copy(acc_sh.at[pl.ds(r0 + off, nrows)],
                        out_hbm.at[c, pl.ds(r0 + off, nrows)])
        off += nrows


def _aggregate_partials(feat, edges, npad):
    _, d = feat.shape
    e = edges.shape[0] // 2
    epw = e // NW
    mesh = plsc.VectorSubcoreMesh(core_axis_name="c", subcore_axis_name="s")
    kern = functools.partial(
        pl.kernel,
        out_type=jax.ShapeDtypeStruct((NC, npad, d), jnp.float32),
        mesh=mesh,
        scratch_types=[
            pltpu.VMEM((epw,), jnp.int32),
            pltpu.VMEM((CHUNK,), jnp.int32),
            pltpu.VMEM((CHUNK,), jnp.int32),
            pltpu.VMEM((CHUNK,), jnp.int32),
            pltpu.VMEM((epw - (epw // CHUNK) * CHUNK,), jnp.int32),
            pltpu.VMEM((3 * CHUNK, d), jnp.float32),
            pltpu.VMEM_SHARED((npad, d), jnp.float32),
        ] + [pltpu.SemaphoreType.DMA] * 9,
        compiler_params=pltpu.CompilerParams(needs_layout_passes=False),
    )(functools.partial(_agg_body, npad=npad, epw=epw, e=e))
    return kern(feat, edges)


def _head_body(p_ref, norm_ref, wgc_ref, bgc_ref, wl_ref, blin_ref,
               out_ref, out2_ref):
    agg = p_ref[0] + p_ref[1]
    rst = jnp.dot(agg, wgc_ref[...], preferred_element_type=jnp.float32)
    rst = rst * norm_ref[...] + bgc_ref[...]
    h = jnp.maximum(rst, 0.0)
    out = lax.dot_general(h, wl_ref[...], (((1,), (1,)), ((), ())),
                          preferred_element_type=jnp.float32) + blin_ref[...]
    out_ref[...] = out
    out2_ref[...] = out


def _head(parts, norm, w_gc, b_gc, w_lin, b_lin, n):
    _, npad, d = parts.shape
    d_out = w_lin.shape[0]
    blk = 2000
    grid = n // blk
    return pl.pallas_call(
        _head_body,
        grid=(grid,),
        in_specs=[
            pl.BlockSpec((2, blk, d), lambda i: (0, i, 0)),
            pl.BlockSpec((blk, 1), lambda i: (i, 0)),
            pl.BlockSpec((d, d), lambda i: (0, 0)),
            pl.BlockSpec((1, d), lambda i: (0, 0)),
            pl.BlockSpec((d_out, d), lambda i: (0, 0)),
            pl.BlockSpec((1, d_out), lambda i: (0, 0)),
        ],
        out_specs=[pl.BlockSpec((blk, d_out), lambda i: (i, 0)),
                   pl.BlockSpec((blk, d_out), lambda i: (i, 0))],
        out_shape=[jax.ShapeDtypeStruct((n, d_out), jnp.float32),
                   jax.ShapeDtypeStruct((n, d_out), jnp.float32)],
    )(parts, norm, w_gc, b_gc, w_lin, b_lin)


def kernel(n_subgraph, e_subgraph, to_fetch, features, W_gc, b_gc, W_lin, b_lin):
    n = n_subgraph.shape[0]

    npad = ((n + NS * 128 - 1) // (NS * 128)) * (NS * 128)
    features_pad = jnp.pad(features, ((0, npad - n), (0, 0)))
    edges_flat = e_subgraph.reshape(-1)
    deg_part = _degree_partials(edges_flat, npad)
    feat, norm = _scaled_features(deg_part, features_pad)
    parts = _aggregate_partials(feat, edges_flat, npad)
    out, out2 = _head(parts, norm, W_gc, b_gc.reshape(1, -1),
                      W_lin, b_lin.reshape(1, -1), n)
    return (out, out2)
